# baseline (speedup 1.0000x reference)
import jax
import jax.numpy as jnp
from jax import lax
from jax.experimental import pallas as pl
from jax.experimental.pallas import tpu as pltpu
from jax.experimental.pallas import tpu_sc as plsc

_K = 8192
_D = 256
_N = 8192
_TB = 256
_NT = _N // _TB

_NC, _NS = 2, 16
_NW = _NC * _NS
_BPW = _N // _NW
_CH = 128
_NCH = _BPW // _CH


def _dist_body(lat_ref, e_ref, e2_ref, idx_ref, oh_ref):
    i = pl.program_id(0)
    blk = lat_ref[...]
    x = jnp.transpose(jnp.reshape(blk, (_D, _TB)))
    xm2 = x * -2.0
    mm = lax.dot_general(xm2, e_ref[...], (((1,), (1,)), ((), ())),
                         preferred_element_type=jnp.float32)
    x2 = jnp.sum(x * x, axis=1, keepdims=True)
    d = (x2 + e2_ref[...]) + mm
    dmin = jnp.min(d, axis=1, keepdims=True)
    iota = lax.broadcasted_iota(jnp.int32, (_TB, _K), 1)
    idx = jnp.min(jnp.where(d == dmin, iota, _K),
                  axis=1, keepdims=True)
    idx_ref[pl.ds(2 * (i % 4), 2), :] = jnp.reshape(idx, (_TB // _CH, _CH))
    oh = (iota == idx).astype(jnp.float32)
    oh_ref[...] = oh


def _distances_argmin(latents, emb, e2):
    return pl.pallas_call(
        _dist_body,
        grid=(_NT,),
        in_specs=[
            pl.BlockSpec((1, _D, 8, 32), lambda i: (i // 4, 0, i % 4, 0)),
            pl.BlockSpec((_K, _D), lambda i: (0, 0)),
            pl.BlockSpec((1, _K), lambda i: (0, 0)),
        ],
        out_specs=[
            pl.BlockSpec((8, _CH), lambda i: (i // 4, 0)),
            pl.BlockSpec((_TB, _K), lambda i: (i, 0)),
        ],
        out_shape=[
            jax.ShapeDtypeStruct((_N // _CH, _CH), jnp.int32),
            jax.ShapeDtypeStruct((_N, _K), jnp.float32),
        ],
        compiler_params=pltpu.CompilerParams(
            dimension_semantics=("arbitrary",),
        ),
    )(latents, emb, e2)


def _gather_body(table_hbm, idx_hbm, zeros_hbm, out_hbm, cnt_hbm,
                 idx_v, rows_v, ones_v, cnt_sh, sem):
    c = lax.axis_index("c")
    s = lax.axis_index("s")
    wid = s * _NC + c
    pltpu.sync_copy(idx_hbm.at[pl.ds(wid * _NCH, _NCH)], idx_v)
    copies = [
        pltpu.async_copy(table_hbm.at[idx_v.at[j]],
                         rows_v.at[pl.ds(j * _CH, _CH)], sem)
        for j in range(_NCH)
    ]
    @pl.when(s == 0)
    def _():
        pltpu.sync_copy(zeros_hbm, cnt_sh)
    for j in range(_CH // 16):
        ones_v[pl.ds(j * 16, 16)] = jnp.full((16,), 1.0, jnp.float32)
    plsc.subcore_barrier()
    for j in range(_NCH):
        pltpu.sync_copy(ones_v, cnt_sh.at[idx_v.at[j]], add=True)
    for cp in copies:
        cp.wait()
    pltpu.sync_copy(rows_v, out_hbm.at[pl.ds(wid * _BPW, _BPW)])
    plsc.subcore_barrier()
    @pl.when(s == 0)
    def _():
        pltpu.sync_copy(cnt_sh, cnt_hbm.at[c])


def _sc_gather_hist(table, idx2d, zeros_k):
    fn = pl.kernel(
        _gather_body,
        mesh=plsc.VectorSubcoreMesh(core_axis_name="c", subcore_axis_name="s"),
        out_type=[
            jax.ShapeDtypeStruct((_N, _D), jnp.float32),
            jax.ShapeDtypeStruct((_NC, _K), jnp.float32),
        ],
        scratch_types=[
            pltpu.VMEM((_NCH, _CH), jnp.int32),
            pltpu.VMEM((_BPW, _D), jnp.float32),
            pltpu.VMEM((_CH,), jnp.float32),
            pltpu.VMEM_SHARED((_K,), jnp.float32),
            pltpu.SemaphoreType.DMA,
        ],
    )
    return fn(table, idx2d, zeros_k)


def _loss_body(q_ref, lat_ref, cnt_ref, qst_ref, loss_ref, perp_ref, acc_ref):
    b = pl.program_id(0)
    lat2 = jnp.reshape(lat_ref[...], (_D, 1024))
    qt = jnp.transpose(q_ref[...])
    dqx = qt - lat2
    qst_ref[...] = jnp.reshape(lat2 + dqx, (1, _D, 32, 32))

    @pl.when(b == 0)
    def _():
        acc_ref[...] = jnp.zeros_like(acc_ref)

    acc_ref[...] += jnp.reshape(jnp.sum(dqx * dqx), (1, 1))

    @pl.when(b == 7)
    def _():
        mse = acc_ref[0, 0] * (1.0 / (_N * _D))
        loss_ref[...] = jnp.reshape(mse + 0.25 * mse, (1, 1))
        cnt = cnt_ref[...]
        p = (cnt[0:1, :] + cnt[1:2, :]) * (1.0 / _N)
        ent = -jnp.sum(p * jnp.log(p + 1e-10))
        perp_ref[...] = jnp.reshape(jnp.exp(ent), (1, 1))


def _loss_perplexity(q, latents, counts):
    return pl.pallas_call(
        _loss_body,
        grid=(8,),
        in_specs=[
            pl.BlockSpec((1024, _D), lambda b: (b, 0)),
            pl.BlockSpec((1, _D, 32, 32), lambda b: (b, 0, 0, 0)),
            pl.BlockSpec((_NC, _K), lambda b: (0, 0)),
        ],
        out_specs=[
            pl.BlockSpec((1, _D, 32, 32), lambda b: (b, 0, 0, 0)),
            pl.BlockSpec((1, 1), lambda b: (0, 0)),
            pl.BlockSpec((1, 1), lambda b: (0, 0)),
        ],
        out_shape=[
            jax.ShapeDtypeStruct((8, _D, 32, 32), jnp.float32),
            jax.ShapeDtypeStruct((1, 1), jnp.float32),
            jax.ShapeDtypeStruct((1, 1), jnp.float32),
        ],
        scratch_shapes=[pltpu.VMEM((1, 1), jnp.float32)],
        compiler_params=pltpu.CompilerParams(
            dimension_semantics=("arbitrary",),
        ),
    )(q, latents, counts)


def kernel(latents, embedding_weight):
    e2 = jnp.sum(embedding_weight * embedding_weight, axis=1)[None, :]
    idx2d, encodings = _distances_argmin(latents, embedding_weight, e2)
    zeros_k = jnp.zeros((_K,), jnp.float32)
    q, counts = _sc_gather_hist(embedding_weight, idx2d, zeros_k)
    quantized_out, loss, perp = _loss_perplexity(q, latents, counts)
    return (loss[0, 0], quantized_out, perp[0, 0], encodings)


# speedup vs baseline: 1.5222x; 1.5222x over previous
import jax
import jax.numpy as jnp
from jax import lax
from jax.experimental import pallas as pl
from jax.experimental.pallas import tpu as pltpu
from jax.experimental.pallas import tpu_sc as plsc

_K = 8192
_D = 256
_N = 8192
_TB = 256
_NT = _N // _TB

_NC, _NS = 2, 16
_NW = _NC * _NS
_BPW = _N // _NW
_CH = 128
_NCH = _BPW // _CH


def _dist_body(x_ref, e_ref, e2_ref, idx_ref, oh_ref):
    i = pl.program_id(0)
    x = x_ref[...]
    xm2 = x * -2.0
    mm = lax.dot_general(xm2, e_ref[...], (((1,), (1,)), ((), ())),
                         preferred_element_type=jnp.float32)
    x2 = jnp.sum(x * x, axis=1, keepdims=True)
    d = (x2 + e2_ref[...]) + mm
    dmin = jnp.min(d, axis=1, keepdims=True)
    iota = lax.broadcasted_iota(jnp.int32, (_TB, _K), 1)
    idx = jnp.min(jnp.where(d == dmin, iota, _K),
                  axis=1, keepdims=True)
    idx_ref[pl.ds(2 * (i % 4), 2), :] = jnp.reshape(idx, (_TB // _CH, _CH))
    oh = (iota == idx).astype(jnp.float32)
    oh_ref[...] = oh


def _distances_argmin(flat, emb, e2):
    return pl.pallas_call(
        _dist_body,
        grid=(_NT,),
        in_specs=[
            pl.BlockSpec((_TB, _D), lambda i: (i, 0)),
            pl.BlockSpec((_K, _D), lambda i: (0, 0)),
            pl.BlockSpec((1, _K), lambda i: (0, 0)),
        ],
        out_specs=[
            pl.BlockSpec((8, _CH), lambda i: (i // 4, 0)),
            pl.BlockSpec((_TB, _K), lambda i: (i, 0)),
        ],
        out_shape=[
            jax.ShapeDtypeStruct((_N // _CH, _CH), jnp.int32),
            jax.ShapeDtypeStruct((_N, _K), jnp.float32),
        ],
        compiler_params=pltpu.CompilerParams(
            dimension_semantics=("arbitrary",),
        ),
    )(flat, emb, e2)


def _gather_body(table_hbm, idx_hbm, zeros_hbm, out_hbm, cnt_hbm,
                 idx_v, rows_v, ones_v, cnt_sh, sem):
    c = lax.axis_index("c")
    s = lax.axis_index("s")
    wid = s * _NC + c
    pltpu.sync_copy(idx_hbm.at[pl.ds(wid * _NCH, _NCH)], idx_v)
    copies = [
        pltpu.async_copy(table_hbm.at[idx_v.at[j]],
                         rows_v.at[pl.ds(j * _CH, _CH)], sem)
        for j in range(_NCH)
    ]
    @pl.when(s == 0)
    def _():
        pltpu.sync_copy(zeros_hbm, cnt_sh)
    for j in range(_CH // 16):
        ones_v[pl.ds(j * 16, 16)] = jnp.full((16,), 1.0, jnp.float32)
    plsc.subcore_barrier()
    for j in range(_NCH):
        pltpu.sync_copy(ones_v, cnt_sh.at[idx_v.at[j]], add=True)
    for cp in copies:
        cp.wait()
    pltpu.sync_copy(rows_v, out_hbm.at[pl.ds(wid * _BPW, _BPW)])
    plsc.subcore_barrier()
    @pl.when(s == 0)
    def _():
        pltpu.sync_copy(cnt_sh, cnt_hbm.at[c])


def _sc_gather_hist(table, idx2d, zeros_k):
    fn = pl.kernel(
        _gather_body,
        mesh=plsc.VectorSubcoreMesh(core_axis_name="c", subcore_axis_name="s"),
        out_type=[
            jax.ShapeDtypeStruct((_N, _D), jnp.float32),
            jax.ShapeDtypeStruct((_NC, _K), jnp.float32),
        ],
        scratch_types=[
            pltpu.VMEM((_NCH, _CH), jnp.int32),
            pltpu.VMEM((_BPW, _D), jnp.float32),
            pltpu.VMEM((_CH,), jnp.float32),
            pltpu.VMEM_SHARED((_K,), jnp.float32),
            pltpu.SemaphoreType.DMA,
        ],
    )
    return fn(table, idx2d, zeros_k)


def _loss_body(q_ref, x_ref, cnt_ref, qst_ref, loss_ref, perp_ref):
    q = q_ref[...]
    x = x_ref[...]
    dqx = q - x
    qst_ref[...] = x + dqx
    mse = jnp.sum(dqx * dqx) * (1.0 / (_N * _D))
    loss_ref[...] = jnp.reshape(mse + 0.25 * mse, (1, 1))
    cnt = cnt_ref[...]
    p = (cnt[0:1, :] + cnt[1:2, :]) * (1.0 / _N)
    ent = -jnp.sum(p * jnp.log(p + 1e-10))
    perp_ref[...] = jnp.reshape(jnp.exp(ent), (1, 1))


def _loss_perplexity(q, flat, counts):
    return pl.pallas_call(
        _loss_body,
        in_specs=[
            pl.BlockSpec((_N, _D), lambda: (0, 0)),
            pl.BlockSpec((_N, _D), lambda: (0, 0)),
            pl.BlockSpec((_NC, _K), lambda: (0, 0)),
        ],
        out_specs=[
            pl.BlockSpec((_N, _D), lambda: (0, 0)),
            pl.BlockSpec((1, 1), lambda: (0, 0)),
            pl.BlockSpec((1, 1), lambda: (0, 0)),
        ],
        out_shape=[
            jax.ShapeDtypeStruct((_N, _D), jnp.float32),
            jax.ShapeDtypeStruct((1, 1), jnp.float32),
            jax.ShapeDtypeStruct((1, 1), jnp.float32),
        ],
    )(q, flat, counts)


def kernel(latents, embedding_weight):
    lat = jnp.transpose(latents, (0, 2, 3, 1))
    flat = lat.reshape(_N, _D)
    e2 = jnp.sum(embedding_weight * embedding_weight, axis=1)[None, :]
    idx2d, encodings = _distances_argmin(flat, embedding_weight, e2)
    zeros_k = jnp.zeros((_K,), jnp.float32)
    q, counts = _sc_gather_hist(embedding_weight, idx2d, zeros_k)
    qst, loss, perp = _loss_perplexity(q, flat, counts)
    quantized_out = jnp.transpose(qst.reshape(8, 32, 32, _D), (0, 3, 1, 2))
    return (loss[0, 0], quantized_out, perp[0, 0], encodings)


# speedup vs baseline: 1.5425x; 1.0133x over previous
import jax
import jax.numpy as jnp
from jax import lax
from jax.experimental import pallas as pl
from jax.experimental.pallas import tpu as pltpu
from jax.experimental.pallas import tpu_sc as plsc

_K = 8192
_D = 256
_N = 8192
_TB = 512
_NT = _N // _TB

_NC, _NS = 2, 16
_NW = _NC * _NS
_BPW = _N // _NW
_CH = 128
_NCH = _BPW // _CH


def _dist_body(x_ref, e_ref, e2_ref, idx_ref, oh_ref):
    i = pl.program_id(0)
    x = x_ref[...]
    xm2 = x * -2.0
    mm = lax.dot_general(xm2, e_ref[...], (((1,), (1,)), ((), ())),
                         preferred_element_type=jnp.float32)
    x2 = jnp.sum(x * x, axis=1, keepdims=True)
    d = (x2 + e2_ref[...]) + mm
    dmin = jnp.min(d, axis=1, keepdims=True)
    iota = lax.broadcasted_iota(jnp.int32, (_TB, _K), 1)
    idx = jnp.min(jnp.where(d == dmin, iota, _K),
                  axis=1, keepdims=True)
    idx_ref[pl.ds((_TB // _CH) * (i % (8 // (_TB // _CH))), _TB // _CH), :] = (
        jnp.reshape(idx, (_TB // _CH, _CH)))
    oh = (iota == idx).astype(jnp.float32)
    oh_ref[...] = oh


def _distances_argmin(flat, emb, e2):
    return pl.pallas_call(
        _dist_body,
        grid=(_NT,),
        in_specs=[
            pl.BlockSpec((_TB, _D), lambda i: (i, 0)),
            pl.BlockSpec((_K, _D), lambda i: (0, 0)),
            pl.BlockSpec((1, _K), lambda i: (0, 0)),
        ],
        out_specs=[
            pl.BlockSpec((8, _CH), lambda i: (i // (8 // (_TB // _CH)), 0)),
            pl.BlockSpec((_TB, _K), lambda i: (i, 0)),
        ],
        out_shape=[
            jax.ShapeDtypeStruct((_N // _CH, _CH), jnp.int32),
            jax.ShapeDtypeStruct((_N, _K), jnp.float32),
        ],
        compiler_params=pltpu.CompilerParams(
            dimension_semantics=("arbitrary",),
        ),
    )(flat, emb, e2)


def _gather_body(table_hbm, idx_hbm, zeros_hbm, out_hbm, cnt_hbm,
                 idx_v, rows_v, ones_v, cnt_sh, sem):
    c = lax.axis_index("c")
    s = lax.axis_index("s")
    wid = s * _NC + c
    pltpu.sync_copy(idx_hbm.at[pl.ds(wid * _NCH, _NCH)], idx_v)
    copies = [
        pltpu.async_copy(table_hbm.at[idx_v.at[j]],
                         rows_v.at[pl.ds(j * _CH, _CH)], sem)
        for j in range(_NCH)
    ]
    @pl.when(s == 0)
    def _():
        pltpu.sync_copy(zeros_hbm, cnt_sh)
    for j in range(_CH // 16):
        ones_v[pl.ds(j * 16, 16)] = jnp.full((16,), 1.0, jnp.float32)
    plsc.subcore_barrier()
    for j in range(_NCH):
        pltpu.sync_copy(ones_v, cnt_sh.at[idx_v.at[j]], add=True)
    for cp in copies:
        cp.wait()
    pltpu.sync_copy(rows_v, out_hbm.at[pl.ds(wid * _BPW, _BPW)])
    plsc.subcore_barrier()
    @pl.when(s == 0)
    def _():
        pltpu.sync_copy(cnt_sh, cnt_hbm.at[c])


def _sc_gather_hist(table, idx2d, zeros_k):
    fn = pl.kernel(
        _gather_body,
        mesh=plsc.VectorSubcoreMesh(core_axis_name="c", subcore_axis_name="s"),
        out_type=[
            jax.ShapeDtypeStruct((_N, _D), jnp.float32),
            jax.ShapeDtypeStruct((_NC, _K), jnp.float32),
        ],
        scratch_types=[
            pltpu.VMEM((_NCH, _CH), jnp.int32),
            pltpu.VMEM((_BPW, _D), jnp.float32),
            pltpu.VMEM((_CH,), jnp.float32),
            pltpu.VMEM_SHARED((_K,), jnp.float32),
            pltpu.SemaphoreType.DMA,
        ],
    )
    return fn(table, idx2d, zeros_k)


def _loss_body(q_ref, x_ref, cnt_ref, qst_ref, loss_ref, perp_ref):
    q = q_ref[...]
    x = x_ref[...]
    dqx = q - x
    qst_ref[...] = x + dqx
    mse = jnp.sum(dqx * dqx) * (1.0 / (_N * _D))
    loss_ref[...] = jnp.reshape(mse + 0.25 * mse, (1, 1))
    cnt = cnt_ref[...]
    p = (cnt[0:1, :] + cnt[1:2, :]) * (1.0 / _N)
    ent = -jnp.sum(p * jnp.log(p + 1e-10))
    perp_ref[...] = jnp.reshape(jnp.exp(ent), (1, 1))


def _loss_perplexity(q, flat, counts):
    return pl.pallas_call(
        _loss_body,
        in_specs=[
            pl.BlockSpec((_N, _D), lambda: (0, 0)),
            pl.BlockSpec((_N, _D), lambda: (0, 0)),
            pl.BlockSpec((_NC, _K), lambda: (0, 0)),
        ],
        out_specs=[
            pl.BlockSpec((_N, _D), lambda: (0, 0)),
            pl.BlockSpec((1, 1), lambda: (0, 0)),
            pl.BlockSpec((1, 1), lambda: (0, 0)),
        ],
        out_shape=[
            jax.ShapeDtypeStruct((_N, _D), jnp.float32),
            jax.ShapeDtypeStruct((1, 1), jnp.float32),
            jax.ShapeDtypeStruct((1, 1), jnp.float32),
        ],
    )(q, flat, counts)


def kernel(latents, embedding_weight):
    lat = jnp.transpose(latents, (0, 2, 3, 1))
    flat = lat.reshape(_N, _D)
    e2 = jnp.sum(embedding_weight * embedding_weight, axis=1)[None, :]
    idx2d, encodings = _distances_argmin(flat, embedding_weight, e2)
    zeros_k = jnp.zeros((_K,), jnp.float32)
    q, counts = _sc_gather_hist(embedding_weight, idx2d, zeros_k)
    qst, loss, perp = _loss_perplexity(q, flat, counts)
    quantized_out = jnp.transpose(qst.reshape(8, 32, 32, _D), (0, 3, 1, 2))
    return (loss[0, 0], quantized_out, perp[0, 0], encodings)


# speedup vs baseline: 1.5556x; 1.0085x over previous
import jax
import jax.numpy as jnp
from jax import lax
from jax.experimental import pallas as pl
from jax.experimental.pallas import tpu as pltpu
from jax.experimental.pallas import tpu_sc as plsc

_K = 8192
_D = 256
_N = 8192
_TB = 512
_NT = _N // _TB

_NC, _NS = 2, 16
_NW = _NC * _NS
_BPW = _N // _NW
_CH = 128
_NCH = _BPW // _CH


def _dist_body(x_ref, e_ref, e2_ref, idx_ref, oh_ref):
    i = pl.program_id(0)
    x = x_ref[...]
    xm2 = x * -2.0
    mm = lax.dot_general(xm2, e_ref[...], (((1,), (1,)), ((), ())),
                         preferred_element_type=jnp.float32)
    x2 = jnp.sum(x * x, axis=1, keepdims=True)
    d = (x2 + e2_ref[...]) + mm
    dmin = jnp.min(d, axis=1, keepdims=True)
    iota = lax.broadcasted_iota(jnp.int32, (_TB, _K), 1)
    idx = jnp.min(jnp.where(d == dmin, iota, _K),
                  axis=1, keepdims=True)
    idx_ref[pl.ds((_TB // _CH) * (i % (8 // (_TB // _CH))), _TB // _CH), :] = (
        jnp.reshape(idx, (_TB // _CH, _CH)))
    oh = (iota == idx).astype(jnp.float32)
    oh_ref[...] = oh


def _distances_argmin(flat, emb, e2):
    return pl.pallas_call(
        _dist_body,
        grid=(_NT,),
        in_specs=[
            pl.BlockSpec((_TB, _D), lambda i: (i, 0)),
            pl.BlockSpec((_K, _D), lambda i: (0, 0)),
            pl.BlockSpec((1, _K), lambda i: (0, 0)),
        ],
        out_specs=[
            pl.BlockSpec((8, _CH), lambda i: (i // (8 // (_TB // _CH)), 0)),
            pl.BlockSpec((_TB, _K), lambda i: (i, 0)),
        ],
        out_shape=[
            jax.ShapeDtypeStruct((_N // _CH, _CH), jnp.int32),
            jax.ShapeDtypeStruct((_N, _K), jnp.float32),
        ],
        compiler_params=pltpu.CompilerParams(
            dimension_semantics=("arbitrary",),
        ),
    )(flat, emb, e2)


def _gather_body(table_hbm, idx_hbm, zeros_hbm, out_hbm, cnt_hbm,
                 idx_v, rows_v, ones_v, cnt_sh, sem):
    c = lax.axis_index("c")
    s = lax.axis_index("s")
    wid = s * _NC + c
    pltpu.sync_copy(idx_hbm.at[pl.ds(wid * _NCH, _NCH)], idx_v)
    copies = [
        pltpu.async_copy(table_hbm.at[idx_v.at[j]],
                         rows_v.at[pl.ds(j * _CH, _CH)], sem)
        for j in range(_NCH)
    ]
    @pl.when(s == 0)
    def _():
        pltpu.sync_copy(zeros_hbm, cnt_sh)
    for j in range(_CH // 16):
        ones_v[pl.ds(j * 16, 16)] = jnp.full((16,), 1.0, jnp.float32)
    plsc.subcore_barrier()
    for j in range(_NCH):
        pltpu.sync_copy(ones_v, cnt_sh.at[idx_v.at[j]], add=True)
    for cp in copies:
        cp.wait()
    pltpu.sync_copy(rows_v, out_hbm.at[pl.ds(wid * _BPW, _BPW)])
    plsc.subcore_barrier()
    @pl.when(s == 0)
    def _():
        pltpu.sync_copy(cnt_sh, cnt_hbm.at[c])


def _sc_gather_hist(table, idx2d, zeros_k):
    fn = pl.kernel(
        _gather_body,
        mesh=plsc.VectorSubcoreMesh(core_axis_name="c", subcore_axis_name="s"),
        out_type=[
            jax.ShapeDtypeStruct((_N, _D), jnp.float32),
            jax.ShapeDtypeStruct((_NC, _K), jnp.float32),
        ],
        scratch_types=[
            pltpu.VMEM((_NCH, _CH), jnp.int32),
            pltpu.VMEM((_BPW, _D), jnp.float32),
            pltpu.VMEM((_CH,), jnp.float32),
            pltpu.VMEM_SHARED((_K,), jnp.float32),
            pltpu.SemaphoreType.DMA,
        ],
    )
    return fn(table, idx2d, zeros_k)


def _loss_body(q_ref, x_ref, cnt_ref, loss_ref, perp_ref, acc_ref):
    b = pl.program_id(0)
    dqx = q_ref[...] - x_ref[...]

    @pl.when(b == 0)
    def _():
        acc_ref[...] = jnp.zeros_like(acc_ref)

    acc_ref[...] += jnp.reshape(jnp.sum(dqx * dqx), (1, 1))

    @pl.when(b == 7)
    def _():
        mse = acc_ref[0, 0] * (1.0 / (_N * _D))
        loss_ref[...] = jnp.reshape(mse + 0.25 * mse, (1, 1))
        cnt = cnt_ref[...]
        p = (cnt[0:1, :] + cnt[1:2, :]) * (1.0 / _N)
        ent = -jnp.sum(p * jnp.log(p + 1e-10))
        perp_ref[...] = jnp.reshape(jnp.exp(ent), (1, 1))


def _loss_perplexity(q, flat, counts):
    return pl.pallas_call(
        _loss_body,
        grid=(8,),
        in_specs=[
            pl.BlockSpec((1024, _D), lambda b: (b, 0)),
            pl.BlockSpec((1024, _D), lambda b: (b, 0)),
            pl.BlockSpec((_NC, _K), lambda b: (0, 0)),
        ],
        out_specs=[
            pl.BlockSpec((1, 1), lambda b: (0, 0)),
            pl.BlockSpec((1, 1), lambda b: (0, 0)),
        ],
        out_shape=[
            jax.ShapeDtypeStruct((1, 1), jnp.float32),
            jax.ShapeDtypeStruct((1, 1), jnp.float32),
        ],
        scratch_shapes=[pltpu.VMEM((1, 1), jnp.float32)],
        compiler_params=pltpu.CompilerParams(
            dimension_semantics=("arbitrary",),
        ),
    )(q, flat, counts)


def kernel(latents, embedding_weight):
    lat = jnp.transpose(latents, (0, 2, 3, 1))
    flat = lat.reshape(_N, _D)
    e2 = jnp.sum(embedding_weight * embedding_weight, axis=1)[None, :]
    idx2d, encodings = _distances_argmin(flat, embedding_weight, e2)
    zeros_k = jnp.zeros((_K,), jnp.float32)
    q, counts = _sc_gather_hist(embedding_weight, idx2d, zeros_k)
    loss, perp = _loss_perplexity(q, flat, counts)
    quantized_out = jnp.transpose(q.reshape(8, 32, 32, _D), (0, 3, 1, 2))
    return (loss[0, 0], quantized_out, perp[0, 0], encodings)
